# Initial kernel scaffold; baseline (speedup 1.0000x reference)
import functools

import jax
import jax.numpy as jnp
from jax import lax
from jax.experimental import pallas as pl
from jax.experimental.pallas import tpu as pltpu
from jax.experimental.pallas import tpu_sc as plsc

D = 64
NC, NS = 2, 16
NW = NC * NS


@functools.partial(jax.jit, static_argnums=(4, 5))
def _lookup_sum(ix, iy, wx, wy, n, c):
    per_w = n // NW
    n_chunks = per_w // c
    mesh = plsc.VectorSubcoreMesh(core_axis_name="c", subcore_axis_name="s")

    def body(ix_hbm, iy_hbm, wx_hbm, wy_hbm, out_hbm,
             idxx_v, idxy_v, bufa_v, bufb_v, sema, semb):
        wid = lax.axis_index("s") * NC + lax.axis_index("c")
        base = wid * per_w

        def chunk(k, carry):
            row0 = base + k * c
            pltpu.sync_copy(ix_hbm.at[pl.ds(row0, c)], idxx_v)
            pltpu.sync_copy(iy_hbm.at[pl.ds(row0, c)], idxy_v)
            cpa = pltpu.async_copy(wx_hbm.at[idxx_v], bufa_v, sema)
            cpb = pltpu.async_copy(wy_hbm.at[idxy_v], bufb_v, semb)
            cpa.wait()
            cpb.wait()

            def add_row(i, carry2):
                for j in range(D // 16):
                    s = pl.ds(j * 16, 16)
                    bufa_v[i, s] = bufa_v[i, s] + bufb_v[i, s]
                return carry2

            lax.fori_loop(0, c, add_row, 0, unroll=2)
            pltpu.sync_copy(bufa_v, out_hbm.at[pl.ds(row0, c)])
            return carry

        lax.fori_loop(0, n_chunks, chunk, 0)

    return pl.kernel(
        body,
        out_type=jax.ShapeDtypeStruct((n, D), jnp.float32),
        mesh=mesh,
        scratch_types=[
            pltpu.VMEM((c,), jnp.int32),
            pltpu.VMEM((c,), jnp.int32),
            pltpu.VMEM((c, D), jnp.float32),
            pltpu.VMEM((c, D), jnp.float32),
            pltpu.SemaphoreType.DMA,
            pltpu.SemaphoreType.DMA,
        ],
    )(ix, iy, wx, wy)


def kernel(x_coord, Wx, Wy):
    b, l, _ = x_coord.shape
    n = b * l
    ix = x_coord[..., 0].reshape(n)
    iy = x_coord[..., 1].reshape(n)
    out = _lookup_sum(ix, iy, Wx, Wy, n, 128)
    return out.reshape(b, l, D)


# speedup vs baseline: 3.6433x; 3.6433x over previous
import functools

import jax
import jax.numpy as jnp
from jax import lax
from jax.experimental import pallas as pl
from jax.experimental.pallas import tpu as pltpu
from jax.experimental.pallas import tpu_sc as plsc

D = 64
NC, NS = 2, 16
NW = NC * NS


@functools.partial(jax.jit, static_argnums=(4, 5))
def _lookup_sum(ix, iy, wx, wy, n, c):
    per_w = n // NW
    n_chunks = per_w // c
    mesh = plsc.VectorSubcoreMesh(core_axis_name="c", subcore_axis_name="s")

    def body(ix_hbm, iy_hbm, wx_hbm, wy_hbm, out_hbm,
             idxx_v, idxy_v, bufa_v, bufb_v, sema, semb):
        wid = lax.axis_index("s") * NC + lax.axis_index("c")
        base = wid * per_w

        def chunk(k, carry):
            row0 = base + k * c
            pltpu.sync_copy(ix_hbm.at[pl.ds(row0, c)], idxx_v)
            pltpu.sync_copy(iy_hbm.at[pl.ds(row0, c)], idxy_v)
            cpa = pltpu.async_copy(wx_hbm.at[idxx_v], bufa_v, sema)
            cpb = pltpu.async_copy(wy_hbm.at[idxy_v], bufb_v, semb)
            cpa.wait()
            cpb.wait()

            def add_row(i, carry2):
                for j in range(D // 16):
                    s = pl.ds(j * 16, 16)
                    bufa_v[i, s] = bufa_v[i, s] + bufb_v[i, s]
                return carry2

            lax.fori_loop(0, c, add_row, 0, unroll=2)
            pltpu.sync_copy(bufa_v, out_hbm.at[pl.ds(row0, c)])
            return carry

        lax.fori_loop(0, n_chunks, chunk, 0)

    return pl.kernel(
        body,
        out_type=jax.ShapeDtypeStruct((n, D), jnp.float32),
        mesh=mesh,
        compiler_params=pltpu.CompilerParams(use_tc_tiling_on_sc=False),
        scratch_types=[
            pltpu.VMEM((c,), jnp.int32),
            pltpu.VMEM((c,), jnp.int32),
            pltpu.VMEM((c, D), jnp.float32),
            pltpu.VMEM((c, D), jnp.float32),
            pltpu.SemaphoreType.DMA,
            pltpu.SemaphoreType.DMA,
        ],
    )(ix, iy, wx, wy)


def kernel(x_coord, Wx, Wy):
    b, l, _ = x_coord.shape
    n = b * l
    ix = x_coord[..., 0].reshape(n)
    iy = x_coord[..., 1].reshape(n)
    out = _lookup_sum(ix, iy, Wx, Wy, n, 128)
    return out.reshape(b, l, D)


# speedup vs baseline: 4.2435x; 1.1648x over previous
import functools

import jax
import jax.numpy as jnp
from jax import lax
from jax.experimental import pallas as pl
from jax.experimental.pallas import tpu as pltpu
from jax.experimental.pallas import tpu_sc as plsc

D = 64
NC, NS = 2, 16
NW = NC * NS


@functools.partial(jax.jit, static_argnums=(4, 5))
def _lookup_sum(ix, iy, wx, wy, n, c):
    per_w = n // NW
    n_chunks = per_w // c
    assert n_chunks % 2 == 0
    mesh = plsc.VectorSubcoreMesh(core_axis_name="c", subcore_axis_name="s")

    def body(ix_hbm, iy_hbm, wx_hbm, wy_hbm, out_hbm,
             ix0, ix1, iy0, iy1, ba0, ba1, bb0, bb1,
             ga0, ga1, gb0, gb1, wb0, wb1):
        wid = lax.axis_index("s") * NC + lax.axis_index("c")
        base = wid * per_w
        idxx = (ix0, ix1)
        idxy = (iy0, iy1)
        bufa = (ba0, ba1)
        bufb = (bb0, bb1)
        ga = (ga0, ga1)
        gb = (gb0, gb1)
        wb = (wb0, wb1)

        def stage_and_fire(k, b):
            row0 = base + k * c
            pltpu.sync_copy(ix_hbm.at[pl.ds(row0, c)], idxx[b])
            pltpu.sync_copy(iy_hbm.at[pl.ds(row0, c)], idxy[b])
            pltpu.async_copy(wx_hbm.at[idxx[b]], bufa[b], ga[b])
            pltpu.async_copy(wy_hbm.at[idxy[b]], bufb[b], gb[b])

        def wait_gathers(b):
            pltpu.make_async_copy(wx_hbm.at[idxx[b]], bufa[b], ga[b]).wait()
            pltpu.make_async_copy(wy_hbm.at[idxy[b]], bufb[b], gb[b]).wait()

        def wait_wb(k, b):
            pltpu.make_async_copy(
                bufa[b], out_hbm.at[pl.ds(base + k * c, c)], wb[b]).wait()

        stage_and_fire(0, 0)

        def pair(k2, carry):
            for b in (0, 1):
                k = 2 * k2 + b
                b1 = 1 - b

                @pl.when(k >= 1)
                def _():
                    wait_wb(k - 1, b1)

                @pl.when(k + 1 < n_chunks)
                def _():
                    stage_and_fire(k + 1, b1)

                wait_gathers(b)

                def add_row(i, carry2):
                    for j in range(D // 16):
                        s = pl.ds(j * 16, 16)
                        bufa[b][i, s] = bufa[b][i, s] + bufb[b][i, s]
                    return carry2

                lax.fori_loop(0, c, add_row, 0, unroll=4)
                pltpu.async_copy(
                    bufa[b], out_hbm.at[pl.ds(base + k * c, c)], wb[b])
            return carry

        lax.fori_loop(0, n_chunks // 2, pair, 0)
        wait_wb(n_chunks - 1, 1)

    return pl.kernel(
        body,
        out_type=jax.ShapeDtypeStruct((n, D), jnp.float32),
        mesh=mesh,
        compiler_params=pltpu.CompilerParams(use_tc_tiling_on_sc=False),
        scratch_types=[
            pltpu.VMEM((c,), jnp.int32),
            pltpu.VMEM((c,), jnp.int32),
            pltpu.VMEM((c,), jnp.int32),
            pltpu.VMEM((c,), jnp.int32),
            pltpu.VMEM((c, D), jnp.float32),
            pltpu.VMEM((c, D), jnp.float32),
            pltpu.VMEM((c, D), jnp.float32),
            pltpu.VMEM((c, D), jnp.float32),
            pltpu.SemaphoreType.DMA,
            pltpu.SemaphoreType.DMA,
            pltpu.SemaphoreType.DMA,
            pltpu.SemaphoreType.DMA,
            pltpu.SemaphoreType.DMA,
            pltpu.SemaphoreType.DMA,
        ],
    )(ix, iy, wx, wy)


def kernel(x_coord, Wx, Wy):
    b, l, _ = x_coord.shape
    n = b * l
    ix = x_coord[..., 0].reshape(n)
    iy = x_coord[..., 1].reshape(n)
    out = _lookup_sum(ix, iy, Wx, Wy, n, 128)
    return out.reshape(b, l, D)
